# baseline (speedup 1.0000x reference)
import functools

import jax
import jax.numpy as jnp
from jax import lax
from jax.experimental import pallas as pl
from jax.experimental.pallas import tpu as pltpu


def _fused_kernel(adj_t_ref, adj_b_ref, x_ref, w1_ref, b1_ref, w2_ref,
                  b2_ref, ot_ref, ob_ref, s1_ref, s2_ref, *, nm, bmh, half):
    i = pl.program_id(0)

    @pl.when(i == 0)
    def _prologue():
        s1_ref[...] = jnp.dot(x_ref[...], w1_ref[...],
                              preferred_element_type=jnp.float32)

    @pl.when(i < nm)
    def _phase1():
        for ref, base in ((adj_t_ref, 0), (adj_b_ref, half)):
            acc = jnp.dot(ref[...], s1_ref[...],
                          preferred_element_type=jnp.float32)
            h = jnp.maximum(acc + b1_ref[...], 0.0)
            s2_ref[pl.ds(base + i * bmh, bmh), :] = jnp.dot(
                h, w2_ref[...], preferred_element_type=jnp.float32)

    @pl.when(i >= nm)
    def _phase2():
        for ref, o_ref in ((adj_t_ref, ot_ref), (adj_b_ref, ob_ref)):
            o = jnp.dot(ref[...], s2_ref[...],
                        preferred_element_type=jnp.float32) + b2_ref[...]
            m = jnp.max(o, axis=1, keepdims=True)
            e = o - m
            lse = jnp.log(jnp.sum(jnp.exp(e), axis=1, keepdims=True))
            o_ref[...] = e - lse


def kernel(x, adj, W1, b1, W2, b2):
    n, nfeat = x.shape
    nhid = W1.shape[1]
    nclass = W2.shape[1]

    bmh = 200 if n % 400 == 0 else n // 2
    half = n // 2
    nm = half // bmh
    nb = half // bmh

    out_t, out_b = pl.pallas_call(
        functools.partial(_fused_kernel, nm=nm, bmh=bmh, half=half),
        grid=(2 * nm,),
        in_specs=[
            pl.BlockSpec((bmh, n), lambda i: (lax.rem(i, nm), 0)),
            pl.BlockSpec((bmh, n), lambda i: (nb + lax.rem(i, nm), 0)),
            pl.BlockSpec((n, nfeat), lambda i: (0, 0)),
            pl.BlockSpec((nfeat, nhid), lambda i: (0, 0)),
            pl.BlockSpec((1, nhid), lambda i: (0, 0)),
            pl.BlockSpec((nhid, nclass), lambda i: (0, 0)),
            pl.BlockSpec((1, nclass), lambda i: (0, 0)),
        ],
        out_specs=[
            pl.BlockSpec((bmh, nclass), lambda i: (jnp.maximum(i - nm, 0), 0)),
            pl.BlockSpec((bmh, nclass), lambda i: (jnp.maximum(i - nm, 0), 0)),
        ],
        out_shape=[
            jax.ShapeDtypeStruct((half, nclass), jnp.float32),
            jax.ShapeDtypeStruct((half, nclass), jnp.float32),
        ],
        scratch_shapes=[
            pltpu.VMEM((n, nhid), jnp.float32),
            pltpu.VMEM((n, nclass), jnp.float32),
        ],
        compiler_params=pltpu.CompilerParams(
            dimension_semantics=("arbitrary",)),
    )(adj, adj, x, W1, b1.reshape(1, nhid), W2, b2.reshape(1, nclass))

    return jnp.concatenate([out_t, out_b], axis=0)


# speedup vs baseline: 1.0385x; 1.0385x over previous
import functools

import jax
import jax.numpy as jnp
from jax.experimental import pallas as pl
from jax.experimental.pallas import tpu as pltpu


def _log_softmax(o):
    m = jnp.max(o, axis=1, keepdims=True)
    e = o - m
    return e - jnp.log(jnp.sum(jnp.exp(e), axis=1, keepdims=True))


def _fused_kernel(adj_ref, x_ref, w1_ref, b1_ref, w2_ref, b2_ref,
                  o_ref, s1_ref, s2_ref, s2b_ref, cache_ref, *, nm, bm, nc):
    i = pl.program_id(0)

    @pl.when(i == 0)
    def _prologue():
        s1_ref[...] = jnp.dot(x_ref[...], w1_ref[...],
                              preferred_element_type=jnp.float32)

    @pl.when(i < nm)
    def _phase1():
        acc = jnp.dot(adj_ref[...], s1_ref[...],
                      preferred_element_type=jnp.float32)
        h = jnp.maximum(acc + b1_ref[...], 0.0)
        s2_ref[pl.ds(i * bm, bm), :] = jnp.dot(
            h, w2_ref[...], preferred_element_type=jnp.float32)

    @pl.when((i >= nm - nc) & (i < nm))
    def _fill_cache():
        cache_ref[pl.ds((i - (nm - nc)) * bm, bm), :] = (
            adj_ref[...].astype(jnp.bfloat16))

    @pl.when(i == nm)
    def _cast_s2():
        s2b_ref[...] = s2_ref[...].astype(jnp.bfloat16)

    @pl.when((i >= nm) & (i < nm + nc))
    def _phase2_cached():
        ab = cache_ref[pl.ds((i - nm) * bm, bm), :]
        o = jnp.dot(ab, s2b_ref[...],
                    preferred_element_type=jnp.float32) + b2_ref[...]
        o_ref[...] = _log_softmax(o)

    @pl.when(i >= nm + nc)
    def _phase2_stream():
        o = jnp.dot(adj_ref[...], s2_ref[...],
                    preferred_element_type=jnp.float32) + b2_ref[...]
        o_ref[...] = _log_softmax(o)


def kernel(x, adj, W1, b1, W2, b2):
    n, nfeat = x.shape
    nhid = W1.shape[1]
    nclass = W2.shape[1]

    if n % 400 == 0:
        bm, nc = 200, 6
    else:
        bm, nc = n // 2, 1
    nm = n // bm

    def adj_idx(i):
        return (jnp.where(i < nm, i,
                          jnp.where(i < nm + nc, nm - 1, i - nm - nc)), 0)

    def out_idx(i):
        return (jnp.where(i < nm, nm - nc,
                          jnp.where(i < nm + nc, i - nc, i - nm - nc)), 0)

    out = pl.pallas_call(
        functools.partial(_fused_kernel, nm=nm, bm=bm, nc=nc),
        grid=(2 * nm,),
        in_specs=[
            pl.BlockSpec((bm, n), adj_idx),
            pl.BlockSpec((n, nfeat), lambda i: (0, 0)),
            pl.BlockSpec((nfeat, nhid), lambda i: (0, 0)),
            pl.BlockSpec((1, nhid), lambda i: (0, 0)),
            pl.BlockSpec((nhid, nclass), lambda i: (0, 0)),
            pl.BlockSpec((1, nclass), lambda i: (0, 0)),
        ],
        out_specs=pl.BlockSpec((bm, nclass), out_idx),
        out_shape=jax.ShapeDtypeStruct((n, nclass), jnp.float32),
        scratch_shapes=[
            pltpu.VMEM((n, nhid), jnp.float32),
            pltpu.VMEM((n, nclass), jnp.float32),
            pltpu.VMEM((n, nclass), jnp.bfloat16),
            pltpu.VMEM((nc * bm, n), jnp.bfloat16),
        ],
        compiler_params=pltpu.CompilerParams(
            dimension_semantics=("arbitrary",)),
    )(adj, x, W1, b1.reshape(1, nhid), W2, b2.reshape(1, nclass))

    return out
